# baseline (speedup 1.0000x reference)
import functools

import jax
import jax.numpy as jnp
from jax import lax
from jax.experimental import pallas as pl
from jax.experimental.pallas import tpu as pltpu
from jax.experimental.pallas import tpu_sc as plsc

VOCAB = 1000
VPAD = 1024
N_TOK = 1024 * 50
NW = 32
NCHUNK = 4
CH = N_TOK // NCHUNK
ROWS_PER_W = CH // NW
G = 16
GROUPS = ROWS_PER_W // G
NBUF = 5
BLK = 1280
NBLK = CH // BLK


def _lse_body(table_ref, out_ref):
    x = table_ref[...]
    m = jnp.max(x, axis=1, keepdims=True)
    s = jnp.sum(jnp.exp(x - m), axis=1, keepdims=True)
    out_ref[...] = m + jnp.log(s)


def _sc_body(table_hbm, idx_hbm, tgt_hbm, lse_hbm,
             out_hbm, part_hbm,
             idx_v, tgt_v, lse_v, r0, r1, r2, r3, r4, acc_v,
             sg0, sg1, sg2, sg3, sg4, so0, so1, so2, so3, so4):
    rows = (r0, r1, r2, r3, r4)
    semg = (sg0, sg1, sg2, sg3, sg4)
    semo = (so0, so1, so2, so3, so4)
    wid = lax.axis_index("s") * 2 + lax.axis_index("c")
    base = wid * ROWS_PER_W

    pltpu.sync_copy(idx_hbm.at[pl.ds(base, ROWS_PER_W)], idx_v)
    pltpu.sync_copy(tgt_hbm.at[pl.ds(base, ROWS_PER_W)], tgt_v)
    pltpu.sync_copy(lse_hbm, lse_v)

    iota = lax.iota(jnp.int32, G)

    def start_gather(g, b):
        idx16 = idx_v[pl.ds(g * G, G)]
        pltpu.async_copy(table_hbm.at[idx16], rows[b], semg[b])

    def wait_gather(b):
        pltpu.make_async_copy(table_hbm.at[iota], rows[b], semg[b]).wait()

    def start_out(g, b):
        pltpu.async_copy(rows[b], out_hbm.at[pl.ds(base + g * G, G)], semo[b])

    def wait_out(b):
        pltpu.make_async_copy(rows[b], out_hbm.at[pl.ds(0, G)],
                              semo[b]).wait()

    for b in range(NBUF):
        start_gather(b, b)

    def body(k, acc):
        for b in range(NBUF):
            g = k * NBUF + b
            wait_gather(b)
            idx16 = idx_v[pl.ds(g * G, G)]
            tg16 = tgt_v[pl.ds(g * G, G)]
            vals = plsc.load_gather(rows[b], [iota, tg16])
            lsev = plsc.load_gather(lse_v, [idx16])
            acc = acc + (vals - lsev)
            start_out(g, b)

            @pl.when(k + 1 < GROUPS // NBUF)
            def _():
                wait_out(b)
                start_gather(g + NBUF, b)

        return acc

    acc = lax.fori_loop(0, GROUPS // NBUF, body, jnp.zeros((G,), jnp.float32))
    for b in range(NBUF):
        wait_out(b)
    acc_v[...] = acc
    pltpu.sync_copy(acc_v, part_hbm.at[pl.ds(wid * G, G)])


def _slice_first_body(in_ref, out_ref):
    out_ref[...] = in_ref[:, :VOCAB]


def _slice_upd_body(big_ref, in_ref, out_ref):
    del big_ref
    out_ref[...] = in_ref[:, :VOCAB]


def _final_body(p0, p1, p2, p3, out_ref):
    tot = (jnp.sum(p0[...]) + jnp.sum(p1[...]) +
           jnp.sum(p2[...]) + jnp.sum(p3[...]))
    out_ref[...] = jnp.full((1, 1), -tot / N_TOK, jnp.float32)


@jax.jit
def kernel(idx, expected, table):
    idx_f = idx.reshape(-1)
    tgt_f = expected.reshape(-1)
    table_pad = jnp.pad(table, ((0, 0), (0, VPAD - VOCAB)))

    lse = pl.pallas_call(
        _lse_body,
        out_shape=jax.ShapeDtypeStruct((VOCAB, 1), jnp.float32),
    )(table)
    lse_f = lse.reshape(-1)

    sc = pl.kernel(
        _sc_body,
        out_type=(
            jax.ShapeDtypeStruct((CH, VPAD), jnp.float32),
            jax.ShapeDtypeStruct((NW * G,), jnp.float32),
        ),
        mesh=plsc.VectorSubcoreMesh(core_axis_name="c", subcore_axis_name="s"),
        compiler_params=pltpu.CompilerParams(needs_layout_passes=False),
        scratch_types=(
            pltpu.VMEM((ROWS_PER_W,), jnp.int32),
            pltpu.VMEM((ROWS_PER_W,), jnp.int32),
            pltpu.VMEM((VOCAB,), jnp.float32),
            pltpu.VMEM((G, VPAD), jnp.float32),
            pltpu.VMEM((G, VPAD), jnp.float32),
            pltpu.VMEM((G, VPAD), jnp.float32),
            pltpu.VMEM((G, VPAD), jnp.float32),
            pltpu.VMEM((G, VPAD), jnp.float32),
            pltpu.VMEM((G,), jnp.float32),
            pltpu.SemaphoreType.DMA,
            pltpu.SemaphoreType.DMA,
            pltpu.SemaphoreType.DMA,
            pltpu.SemaphoreType.DMA,
            pltpu.SemaphoreType.DMA,
            pltpu.SemaphoreType.DMA,
            pltpu.SemaphoreType.DMA,
            pltpu.SemaphoreType.DMA,
            pltpu.SemaphoreType.DMA,
            pltpu.SemaphoreType.DMA,
        ),
    )

    chunks = []
    parts = []
    for c in range(NCHUNK):
        o, p = sc(table_pad,
                  lax.slice(idx_f, (c * CH,), ((c + 1) * CH,)),
                  lax.slice(tgt_f, (c * CH,), ((c + 1) * CH,)),
                  lse_f)
        chunks.append(o)
        parts.append(p)

    out = pl.pallas_call(
        _slice_first_body,
        grid=(NBLK,),
        in_specs=[pl.BlockSpec((BLK, VPAD), lambda i: (i, 0))],
        out_specs=pl.BlockSpec((BLK, VOCAB), lambda i: (i, 0)),
        out_shape=jax.ShapeDtypeStruct((N_TOK, VOCAB), jnp.float32),
    )(chunks[0])

    for c in range(1, NCHUNK):
        out = pl.pallas_call(
            _slice_upd_body,
            grid=(NBLK,),
            in_specs=[
                pl.BlockSpec(memory_space=pl.ANY),
                pl.BlockSpec((BLK, VPAD), lambda i: (i, 0)),
            ],
            out_specs=pl.BlockSpec(
                (BLK, VOCAB), lambda i, c=c: (i + c * NBLK, 0)),
            out_shape=jax.ShapeDtypeStruct((N_TOK, VOCAB), jnp.float32),
            input_output_aliases={0: 0},
        )(out, chunks[c])

    cost = pl.pallas_call(
        _final_body,
        out_shape=jax.ShapeDtypeStruct((1, 1), jnp.float32),
    )(*parts)

    return (out, cost[0, 0])
